# baseline (speedup 1.0000x reference)
import functools

import jax
import jax.numpy as jnp
from jax import lax
from jax.experimental import pallas as pl
from jax.experimental.pallas import tpu as pltpu
from jax.experimental.pallas import tpu_sc as plsc

_DIM_IN = 1000
_D = 16
_B = 16384 * 50
_NC = 2
_NS = 16
_NW = _NC * _NS
_BPW = _B // _NW
_C = 2560
_NCHUNK = _BPW // _C
_L = 16


def _body(x_hbm, table_hbm, out_hbm, xbuf, idxbuf, rows, sem):
    wid = lax.axis_index("s") * _NC + lax.axis_index("c")
    base = wid * _BPW
    ii = lax.iota(jnp.int32, _L)

    def chunk(c, carry):
        row0 = base + c * _C
        pltpu.sync_copy(x_hbm.at[pl.ds(2 * row0, 2 * _C)], xbuf)

        def idx_step(j, carry2):
            off = 2 * _L * j + 2 * ii
            ev = plsc.load_gather(xbuf, [off])
            od = plsc.load_gather(xbuf, [off + 1])
            idxbuf[pl.ds(j * _L, _L)] = od * _DIM_IN + ev
            return carry2

        lax.fori_loop(0, _C // _L, idx_step, 0)
        pltpu.async_copy(table_hbm.at[idxbuf], rows, sem).wait()
        pltpu.sync_copy(rows, out_hbm.at[pl.ds(row0, _C)])
        return carry

    lax.fori_loop(0, _NCHUNK, chunk, 0)


def kernel(x, table):
    mesh = plsc.VectorSubcoreMesh(core_axis_name="c", subcore_axis_name="s")
    run = functools.partial(
        pl.kernel,
        mesh=mesh,
        compiler_params=pltpu.CompilerParams(
            needs_layout_passes=False, use_tc_tiling_on_sc=False),
        out_type=jax.ShapeDtypeStruct((_B, _D), jnp.float32),
        scratch_types=[
            pltpu.VMEM((2 * _C,), jnp.int32),
            pltpu.VMEM((_C,), jnp.int32),
            pltpu.VMEM((_C, _D), jnp.float32),
            pltpu.SemaphoreType.DMA,
        ],
    )(_body)
    out = run(x.reshape(-1), table)
    return out.reshape(16384, 50, _D)
